# Initial kernel scaffold; baseline (speedup 1.0000x reference)
import jax
import jax.numpy as jnp
from jax.experimental import pallas as pl


def kernel(x, l1_w, l1_b, l1_g, l1_beta, l2_w, l2_b, l2_g, l2_beta):
    raise NotImplementedError("write your pallas kernel here")


# speedup vs baseline: 1.6651x; 1.6651x over previous
import functools

import jax
import jax.numpy as jnp
from jax.experimental import pallas as pl
from jax.experimental.pallas import tpu as pltpu

BN_EPS = 1e-5
KSIZE = 3
PAD = 1
VMEM_LIMIT_BYTES = 64 * 1024 * 1024


def _round_up(x, m):
    return (x + m - 1) // m * m


def _conv_from_ref(slice_fn, w_ref, *, tap_offsets, base):
    acc = None
    for t, off in enumerate(tap_offsets):
        lhs = slice_fn(base + off)
        part = jnp.dot(lhs, w_ref[t], preferred_element_type=jnp.float32)
        acc = part if acc is None else acc + part
    return acc


def _masked_stats(y, sum_ref, ssq_ref, *, m_rows, w_pad, w_out):
    col = jax.lax.broadcasted_iota(jnp.int32, (m_rows, 1), 0) % w_pad
    yv = jnp.where(col < w_out, y, 0.0)
    s = jnp.sum(yv, axis=0, keepdims=True)
    q = jnp.sum(yv * yv, axis=0, keepdims=True)
    sum_ref[...] = jnp.broadcast_to(s[None], sum_ref.shape)
    ssq_ref[...] = jnp.broadcast_to(q[None], ssq_ref.shape)


def _conv_stats_kernel(x_ref, w_ref, y_ref, sum_ref, ssq_ref, *,
                       m_rows, w_pad, w_out, tap_offsets):
    acc = _conv_from_ref(lambda o: x_ref[0, pl.ds(o, m_rows), :], w_ref,
                         tap_offsets=tap_offsets, base=0)
    _masked_stats(acc, sum_ref, ssq_ref, m_rows=m_rows, w_pad=w_pad,
                  w_out=w_out)
    y_ref[0] = acc.astype(y_ref.dtype)


def _bn_conv_stats_kernel(y1_ref, a_ref, c_ref, w_ref,
                          y2_ref, sum_ref, ssq_ref, scratch_ref, *,
                          m_rows, w_pad, w_out, tap_offsets,
                          s_off, s_rows):
    z = jnp.maximum(y1_ref[0].astype(jnp.float32) * a_ref[...] + c_ref[...],
                    0.0)
    col = jax.lax.broadcasted_iota(jnp.int32, (m_rows, 1), 0) % w_pad
    z = jnp.where(col < w_out, z, 0.0)
    scratch_ref[pl.ds(0, s_off), :] = jnp.zeros(
        (s_off, z.shape[1]), scratch_ref.dtype)
    scratch_ref[pl.ds(s_off + m_rows, s_rows - s_off - m_rows), :] = (
        jnp.zeros((s_rows - s_off - m_rows, z.shape[1]), scratch_ref.dtype))
    scratch_ref[pl.ds(s_off, m_rows), :] = z.astype(scratch_ref.dtype)
    acc = _conv_from_ref(lambda o: scratch_ref[pl.ds(o, m_rows), :], w_ref,
                         tap_offsets=tap_offsets,
                         base=s_off - (w_pad + 1))
    _masked_stats(acc, sum_ref, ssq_ref, m_rows=m_rows, w_pad=w_pad,
                  w_out=w_out)
    y2_ref[0] = acc.astype(y2_ref.dtype)


def _bn_relu_kernel(y_ref, a_ref, c_ref, o_ref):
    o_ref[0] = jnp.maximum(
        y_ref[0].astype(jnp.float32) * a_ref[...] + c_ref[...], 0.0)


def _fold_bn(sums, ssqs, gamma, beta, count, cout):
    ch_sum = jnp.sum(sums[:, 0, :], axis=0)
    ch_ssq = jnp.sum(ssqs[:, 0, :], axis=0)
    mean = ch_sum / count
    var = jnp.maximum(ch_ssq / count - mean * mean, 0.0)
    scale = gamma * jax.lax.rsqrt(var + BN_EPS)
    a = scale.reshape(1, cout)
    c = (beta - mean * scale).reshape(1, cout)
    return a, c


def _weight_taps(weight):
    w = jnp.transpose(weight, (2, 3, 1, 0))
    k = weight.shape[-1]
    return w.reshape(k * k, weight.shape[1], weight.shape[0]).astype(
        jnp.bfloat16)


def kernel(x, l1_w, l1_b, l1_g, l1_beta, l2_w, l2_b, l2_g, l2_beta):
    del l1_b, l2_b
    n, cin, h, w = x.shape
    mid = l1_w.shape[0]
    cout = l2_w.shape[0]
    h_pad, w_pad = h + 2 * PAD, w + 2 * PAD
    h_out, w_out = h_pad - KSIZE + 1, w_pad - KSIZE + 1
    m_rows = h_out * w_pad
    p_in = _round_up(h_pad * w_pad + KSIZE - 1, 16)
    tap_offsets = tuple(kh * w_pad + kw
                        for kh in range(KSIZE) for kw in range(KSIZE))
    s_off = 80
    s_rows = _round_up(s_off - (w_pad + 1) + tap_offsets[-1] + m_rows, 16)
    count = n * h_out * w_out

    xt = jnp.transpose(x, (0, 2, 3, 1))
    xp = jnp.pad(xt, ((0, 0), (PAD, PAD), (PAD, PAD), (0, 0)))
    x_flat = xp.reshape(n, h_pad * w_pad, cin)
    x_flat = jnp.pad(x_flat, ((0, 0), (0, p_in - h_pad * w_pad), (0, 0)))
    x_flat = x_flat.astype(jnp.bfloat16)
    w1 = _weight_taps(l1_w)
    w2 = _weight_taps(l2_w)

    conv_flops = 2 * n * m_rows * KSIZE * KSIZE * cin * mid
    grid = (n,)
    stats_specs = [
        pl.BlockSpec((1, 8, mid), lambda i: (i, 0, 0)),
        pl.BlockSpec((1, 8, mid), lambda i: (i, 0, 0)),
    ]

    y1, s1, q1 = pl.pallas_call(
        functools.partial(_conv_stats_kernel, m_rows=m_rows, w_pad=w_pad,
                          w_out=w_out, tap_offsets=tap_offsets),
        out_shape=(
            jax.ShapeDtypeStruct((n, m_rows, mid), jnp.bfloat16),
            jax.ShapeDtypeStruct((n, 8, mid), jnp.float32),
            jax.ShapeDtypeStruct((n, 8, mid), jnp.float32),
        ),
        grid_spec=pltpu.PrefetchScalarGridSpec(
            num_scalar_prefetch=0,
            grid=grid,
            in_specs=[
                pl.BlockSpec((1, p_in, cin), lambda i: (i, 0, 0)),
                pl.BlockSpec((KSIZE * KSIZE, cin, mid), lambda i: (0, 0, 0)),
            ],
            out_specs=[pl.BlockSpec((1, m_rows, mid), lambda i: (i, 0, 0))]
            + stats_specs,
        ),
        compiler_params=pltpu.CompilerParams(
            dimension_semantics=("parallel",),
            vmem_limit_bytes=VMEM_LIMIT_BYTES,
        ),
        cost_estimate=pl.CostEstimate(
            flops=conv_flops, transcendentals=0,
            bytes_accessed=2 * (n * p_in * cin + n * m_rows * mid)),
    )(x_flat, w1)

    a1, c1 = _fold_bn(s1, q1, l1_g, l1_beta, count, mid)

    y2, s2, q2 = pl.pallas_call(
        functools.partial(_bn_conv_stats_kernel, m_rows=m_rows, w_pad=w_pad,
                          w_out=w_out, tap_offsets=tap_offsets,
                          s_off=s_off, s_rows=s_rows),
        out_shape=(
            jax.ShapeDtypeStruct((n, m_rows, cout), jnp.bfloat16),
            jax.ShapeDtypeStruct((n, 8, cout), jnp.float32),
            jax.ShapeDtypeStruct((n, 8, cout), jnp.float32),
        ),
        grid_spec=pltpu.PrefetchScalarGridSpec(
            num_scalar_prefetch=0,
            grid=grid,
            in_specs=[
                pl.BlockSpec((1, m_rows, mid), lambda i: (i, 0, 0)),
                pl.BlockSpec((1, mid), lambda i: (0, 0)),
                pl.BlockSpec((1, mid), lambda i: (0, 0)),
                pl.BlockSpec((KSIZE * KSIZE, mid, cout), lambda i: (0, 0, 0)),
            ],
            out_specs=[pl.BlockSpec((1, m_rows, cout), lambda i: (i, 0, 0))]
            + stats_specs,
            scratch_shapes=[pltpu.VMEM((s_rows, mid), jnp.bfloat16)],
        ),
        compiler_params=pltpu.CompilerParams(
            dimension_semantics=("parallel",),
            vmem_limit_bytes=VMEM_LIMIT_BYTES,
        ),
        cost_estimate=pl.CostEstimate(
            flops=conv_flops, transcendentals=0,
            bytes_accessed=2 * (n * m_rows * mid + n * m_rows * cout)),
    )(y1, a1, c1, w2)

    a2, c2 = _fold_bn(s2, q2, l2_g, l2_beta, count, cout)

    out_flat = pl.pallas_call(
        _bn_relu_kernel,
        out_shape=jax.ShapeDtypeStruct((n, m_rows, cout), jnp.float32),
        grid_spec=pltpu.PrefetchScalarGridSpec(
            num_scalar_prefetch=0,
            grid=grid,
            in_specs=[
                pl.BlockSpec((1, m_rows, cout), lambda i: (i, 0, 0)),
                pl.BlockSpec((1, cout), lambda i: (0, 0)),
                pl.BlockSpec((1, cout), lambda i: (0, 0)),
            ],
            out_specs=pl.BlockSpec((1, m_rows, cout), lambda i: (i, 0, 0)),
        ),
        compiler_params=pltpu.CompilerParams(
            dimension_semantics=("parallel",),
            vmem_limit_bytes=VMEM_LIMIT_BYTES,
        ),
        cost_estimate=pl.CostEstimate(
            flops=2 * n * m_rows * cout, transcendentals=0,
            bytes_accessed=6 * n * m_rows * cout),
    )(y2, a2, c2)

    out = out_flat.reshape(n, h_out, w_pad, cout)[:, :, :w_out, :]
    return jnp.transpose(out, (0, 3, 1, 2))
